# Initial kernel scaffold; baseline (speedup 1.0000x reference)
import jax
import jax.numpy as jnp
from jax.experimental import pallas as pl


def kernel(feats, edge_index, node_graph_ids, gcn1_W, gcn1_b, gcn1_bn_g, gcn1_bn_b, gcn2_0_W, gcn2_0_b, gcn2_0_bn_g, gcn2_0_bn_b, gcn2_1_W, gcn2_1_b, gcn2_1_bn_g, gcn2_1_bn_b, aw_W, aw_b, fc1_W, fc1_b, bn1_g, bn1_b, lin0_W, lin0_b, bnl0_g, bnl0_b, fc2_W, fc2_b):
    raise NotImplementedError("write your pallas kernel here")


# speedup vs baseline: 6.4973x; 6.4973x over previous
import functools

import jax
import jax.numpy as jnp
from jax import lax
from jax.experimental import pallas as pl
from jax.experimental.pallas import tpu as pltpu
from jax.experimental.pallas import tpu_sc as plsc

N = 50000
E = 1600000
G = 1024
IN = 38
INP = 64
H = 128
OUT = 67
EPS = 1e-5

BLK = 2000
NBLK = N // BLK

DEGP = 50048
DEGB = 2 * DEGP


def _vsmesh():
    return plsc.VectorSubcoreMesh(core_axis_name="c", subcore_axis_name="s")


_DEG_EPC = E // 2
_DEG_EPT = _DEG_EPC // 16
_DEG_CH = 4096
_DEG_NCH = _DEG_EPT // _DEG_CH
_DEG_TAIL = _DEG_EPT - _DEG_NCH * _DEG_CH
_DEG_TAILP = 896
_DEG_SL = DEGB // 16


@functools.partial(
    pl.kernel,
    out_type=jax.ShapeDtypeStruct((2 * DEGB,), jnp.float32),
    mesh=_vsmesh(),
    scratch_types=[
        pltpu.VMEM((_DEG_CH,), jnp.int32),
        pltpu.VMEM((128,), jnp.int32),
        pltpu.VMEM((128,), jnp.float32),
        pltpu.VMEM((_DEG_SL,), jnp.float32),
        pltpu.VMEM_SHARED((DEGB,), jnp.float32),
    ],
)
def _deg_kernel(src_h, dst_h, out_h, idxb, idxg, onesb, stage, acc):
    c = lax.axis_index("c")
    s = lax.axis_index("s")
    iota16 = lax.iota(jnp.int32, 16)
    ones16 = jnp.ones((16,), jnp.float32)
    zeros16 = jnp.zeros((16,), jnp.float32)
    for j in range(8):
        onesb[pl.ds(j * 16, 16)] = ones16

    def zbody(j, _):
        stage[pl.ds(j * 16, 16)] = zeros16
        return 0
    lax.fori_loop(0, _DEG_SL // 16, zbody, 0)
    pltpu.sync_copy(stage, acc.at[pl.ds(s * _DEG_SL, _DEG_SL)])
    plsc.subcore_barrier()
    ebase = c * _DEG_EPC + s * _DEG_EPT

    def do_groups(n_groups, off):
        def gbody(g, _):
            def cb(j, _2):
                v = idxb[pl.ds(g * 128 + j * 16, 16)]
                idxg[pl.ds(j * 16, 16)] = v + off
                return 0
            lax.fori_loop(0, 8, cb, 0)
            pltpu.sync_copy(onesb, acc.at[idxg], add=True)
            return 0
        lax.fori_loop(0, n_groups, gbody, 0)

    def chbody(ch, _):
        cb0 = ebase + ch * _DEG_CH
        pltpu.sync_copy(src_h.at[pl.ds(cb0, _DEG_CH)], idxb)
        do_groups(_DEG_CH // 128, 0)
        pltpu.sync_copy(dst_h.at[pl.ds(cb0, _DEG_CH)], idxb)
        do_groups(_DEG_CH // 128, DEGP)
        return 0
    lax.fori_loop(0, _DEG_NCH, chbody, 0)

    tb = ebase + _DEG_NCH * _DEG_CH
    for arr_h, off in ((src_h, 0), (dst_h, DEGP)):
        pltpu.sync_copy(arr_h.at[pl.ds(tb, _DEG_TAIL)],
                        idxb.at[pl.ds(0, _DEG_TAIL)])
        for j in range((_DEG_TAILP - _DEG_TAIL) // 16):
            idxb[pl.ds(_DEG_TAIL + j * 16, 16)] = N + iota16
        do_groups(_DEG_TAILP // 128, off)

    plsc.subcore_barrier()
    pltpu.sync_copy(acc.at[pl.ds(s * _DEG_SL, _DEG_SL)], stage)
    pltpu.sync_copy(stage, out_h.at[pl.ds(c * DEGB + s * _DEG_SL, _DEG_SL)])


_SC_EPT = E // 16
_SC_CH = 4096
_SC_GRP = _SC_CH // 128
_SC_NCH = _SC_EPT // _SC_CH
_SC_TAIL = _SC_EPT - _SC_NCH * _SC_CH
_SC_TGRP = _SC_TAIL // 128
_SC_TREM = _SC_TAIL - _SC_TGRP * 128
NP2 = 50048
_SC_WRT = NP2 // 16


def _make_edge_scatter(COLW):
    @functools.partial(
        pl.kernel,
        out_type=[jax.ShapeDtypeStruct((NP2, COLW), jnp.float32)
                  for _ in range(4)],
        mesh=_vsmesh(),
        scratch_types=[
            pltpu.VMEM((_SC_CH,), jnp.int32),
            pltpu.VMEM((_SC_CH,), jnp.int32),
            pltpu.VMEM((128,), jnp.int32),
            pltpu.VMEM((128,), jnp.int32),
            pltpu.VMEM((128,), jnp.int32),
            pltpu.VMEM((128,), jnp.int32),
            pltpu.VMEM((_SC_TREM,), jnp.int32),
            pltpu.VMEM((_SC_TREM,), jnp.int32),
            pltpu.VMEM((128, COLW), jnp.float32),
            pltpu.VMEM((128, COLW), jnp.float32),
            pltpu.SemaphoreType.DMA,
            pltpu.SemaphoreType.DMA,
            pltpu.VMEM_SHARED((NP2, COLW), jnp.float32),
        ],
        compiler_params=pltpu.CompilerParams(use_tc_tiling_on_sc=False),
    )
    def edge_scatter(src_h, dst_h, t0_h, t1_h, t2_h, t3_h,
                     o0_h, o1_h, o2_h, o3_h,
                     srcb, dstb, is0, is1, id0, id1, ist, idt,
                     rows0, rows1, gsem0, gsem1, acc):
        c = lax.axis_index("c")
        s = lax.axis_index("s")
        zeros16 = jnp.zeros((16,), jnp.float32)
        tabs = (t0_h, t1_h, t2_h, t3_h)
        outs = (o0_h, o1_h, o2_h, o3_h)
        ebase = s * _SC_EPT

        def copy_idx(bufref, dstref, g, n128=8):
            def cb(j, _):
                dstref[pl.ds(j * 16, 16)] = bufref[pl.ds(g * 128 + j * 16, 16)]
                return 0
            lax.fori_loop(0, n128, cb, 0)

        def edges(tab):
            def chunk(cb0, _):
                pltpu.sync_copy(src_h.at[pl.ds(cb0, _SC_CH)], srcb)
                pltpu.sync_copy(dst_h.at[pl.ds(cb0, _SC_CH)], dstb)
                copy_idx(srcb, is0, 0)
                copy_idx(dstb, id0, 0)
                pltpu.async_copy(tab.at[is0], rows0, gsem0)

                def kbody(k, _2):
                    g0 = 2 * k
                    copy_idx(srcb, is1, g0 + 1)
                    copy_idx(dstb, id1, g0 + 1)
                    pltpu.async_copy(tab.at[is1], rows1, gsem1)
                    pltpu.make_async_copy(tab.at[is0], rows0, gsem0).wait()
                    pltpu.sync_copy(rows0, acc.at[id0], add=True)

                    @pl.when(k < _SC_GRP // 2 - 1)
                    def _():
                        copy_idx(srcb, is0, g0 + 2)
                        copy_idx(dstb, id0, g0 + 2)
                        pltpu.async_copy(tab.at[is0], rows0, gsem0)
                    pltpu.make_async_copy(tab.at[is1], rows1, gsem1).wait()
                    pltpu.sync_copy(rows1, acc.at[id1], add=True)
                    return 0
                lax.fori_loop(0, _SC_GRP // 2, kbody, 0)
                return 0
            lax.fori_loop(0, _SC_NCH, lambda ch, x: chunk(ebase + ch * _SC_CH, x), 0)

            tb = ebase + _SC_NCH * _SC_CH
            pltpu.sync_copy(src_h.at[pl.ds(tb, _SC_TAIL)],
                            srcb.at[pl.ds(0, _SC_TAIL)])
            pltpu.sync_copy(dst_h.at[pl.ds(tb, _SC_TAIL)],
                            dstb.at[pl.ds(0, _SC_TAIL)])

            def tbody(g, _):
                copy_idx(srcb, is0, g)
                copy_idx(dstb, id0, g)
                pltpu.async_copy(tab.at[is0], rows0, gsem0).wait()
                pltpu.sync_copy(rows0, acc.at[id0], add=True)
                return 0
            lax.fori_loop(0, _SC_TGRP, tbody, 0)
            for j in range(_SC_TREM // 16):
                ist[pl.ds(j * 16, 16)] = srcb[pl.ds(_SC_TGRP * 128 + j * 16, 16)]
                idt[pl.ds(j * 16, 16)] = dstb[pl.ds(_SC_TGRP * 128 + j * 16, 16)]
            pltpu.async_copy(tab.at[ist], rows0.at[pl.ds(0, _SC_TREM)],
                             gsem0).wait()
            pltpu.sync_copy(rows0.at[pl.ds(0, _SC_TREM)], acc.at[idt], add=True)

        def writeout(out_h):
            woff = 0
            while woff < _SC_WRT:
                sz = min(128, _SC_WRT - woff)
                pltpu.sync_copy(acc.at[pl.ds(s * _SC_WRT + woff, sz)],
                                rows0.at[pl.ds(0, sz)])
                pltpu.sync_copy(rows0.at[pl.ds(0, sz)],
                                out_h.at[pl.ds(s * _SC_WRT + woff, sz)])
                woff += sz

        for p in range(2):
            def zbody(i, _):
                for j in range(COLW // 16):
                    rows0[i, pl.ds(j * 16, 16)] = zeros16
                return 0
            lax.fori_loop(0, 128, zbody, 0)
            zoff = 0
            while zoff < _SC_WRT:
                sz = min(128, _SC_WRT - zoff)
                pltpu.sync_copy(rows0.at[pl.ds(0, sz)],
                                acc.at[pl.ds(s * _SC_WRT + zoff, sz)])
                zoff += sz
            plsc.subcore_barrier()

            @pl.when(c == 0)
            def _():
                edges(tabs[2 * p])

            @pl.when(c == 1)
            def _():
                edges(tabs[2 * p + 1])
            plsc.subcore_barrier()

            @pl.when(c == 0)
            def _():
                writeout(outs[2 * p])

            @pl.when(c == 1)
            def _():
                writeout(outs[2 * p + 1])
            plsc.subcore_barrier()

    return edge_scatter


_edge_scatter16 = _make_edge_scatter(16)
_edge_scatter32 = _make_edge_scatter(32)


def _k0_body(f_ref, po0, po1, pi0, pi1,
             t0_ref, t1_ref, t2_ref, t3_ref, ns_ref, nd_ref):
    dout = po0[...] + po1[...]
    din = pi0[...] + pi1[...]
    ns = jnp.where(dout > 0, lax.rsqrt(jnp.maximum(dout, 1.0)), 0.0)
    nd = jnp.where(din > 0, lax.rsqrt(jnp.maximum(din, 1.0)), 0.0)
    ns_ref[...] = ns
    nd_ref[...] = nd
    t = f_ref[...] * ns
    cw = INP // 4
    for q, r in enumerate((t0_ref, t1_ref, t2_ref, t3_ref)):
        r[...] = t[:, q * cw:(q + 1) * cw]


_k0 = pl.pallas_call(
    _k0_body,
    grid=(NBLK,),
    in_specs=[
        pl.BlockSpec((BLK, INP), lambda i: (i, 0)),
        pl.BlockSpec((BLK, 1), lambda i: (i, 0)),
        pl.BlockSpec((BLK, 1), lambda i: (i, 0)),
        pl.BlockSpec((BLK, 1), lambda i: (i, 0)),
        pl.BlockSpec((BLK, 1), lambda i: (i, 0)),
    ],
    out_specs=[pl.BlockSpec((BLK, INP // 4), lambda i: (i, 0))] * 4
    + [
        pl.BlockSpec((BLK, 1), lambda i: (i, 0)),
        pl.BlockSpec((BLK, 1), lambda i: (i, 0)),
    ],
    out_shape=[jax.ShapeDtypeStruct((N, INP // 4), jnp.float32)] * 4
    + [
        jax.ShapeDtypeStruct((N, 1), jnp.float32),
        jax.ShapeDtypeStruct((N, 1), jnp.float32),
    ],
)


def _ka_body(u0, u1, u2, u3, nd_ref, w_ref, b_ref, z_ref, st_ref):
    u = jnp.concatenate([u0[...], u1[...], u2[...], u3[...]], axis=1)
    z = jnp.dot(u * nd_ref[...], w_ref[...],
                preferred_element_type=jnp.float32) + b_ref[...]
    z_ref[...] = z

    @pl.when(pl.program_id(0) == 0)
    def _():
        st_ref[...] = jnp.zeros((2, H), jnp.float32)

    st_ref[...] += jnp.concatenate(
        [jnp.sum(z, 0, keepdims=True), jnp.sum(z * z, 0, keepdims=True)], 0)


def _make_ka(Wd):
    cw = Wd // 4
    return pl.pallas_call(
        _ka_body,
        grid=(NBLK,),
        in_specs=[pl.BlockSpec((BLK, cw), lambda i: (i, 0))] * 4
        + [
            pl.BlockSpec((BLK, 1), lambda i: (i, 0)),
            pl.BlockSpec((Wd, H), lambda i: (0, 0)),
            pl.BlockSpec((1, H), lambda i: (0, 0)),
        ],
        out_specs=[
            pl.BlockSpec((BLK, H), lambda i: (i, 0)),
            pl.BlockSpec((2, H), lambda i: (0, 0)),
        ],
        out_shape=[
            jax.ShapeDtypeStruct((N, H), jnp.float32),
            jax.ShapeDtypeStruct((2, H), jnp.float32),
        ],
    )


_ka64 = _make_ka(INP)
_ka128 = _make_ka(H)


def _bn_coeffs(st, g, bb):
    mu = st[0:1, :] * (1.0 / N)
    var = st[1:2, :] * (1.0 / N) - mu * mu
    a = g * lax.rsqrt(var + EPS)
    cc = bb - mu * a
    return a, cc


def _kb_mid_body(z_ref, st_ref, ns_ref, g_ref, bb_ref,
                 t0_ref, t1_ref, t2_ref, t3_ref):
    a, cc = _bn_coeffs(st_ref[...], g_ref[...], bb_ref[...])
    y = jnp.maximum(z_ref[...] * a + cc, 0.0)
    t = y * ns_ref[...]
    cw = H // 4
    for q, r in enumerate((t0_ref, t1_ref, t2_ref, t3_ref)):
        r[...] = t[:, q * cw:(q + 1) * cw]


_kb_mid = pl.pallas_call(
    _kb_mid_body,
    grid=(NBLK,),
    in_specs=[
        pl.BlockSpec((BLK, H), lambda i: (i, 0)),
        pl.BlockSpec((2, H), lambda i: (0, 0)),
        pl.BlockSpec((BLK, 1), lambda i: (i, 0)),
        pl.BlockSpec((1, H), lambda i: (0, 0)),
        pl.BlockSpec((1, H), lambda i: (0, 0)),
    ],
    out_specs=[pl.BlockSpec((BLK, H // 4), lambda i: (i, 0))] * 4,
    out_shape=[jax.ShapeDtypeStruct((N, H // 4), jnp.float32)] * 4,
)


def _kb_fin_body(z_ref, st_ref, g_ref, bb_ref, aww_ref, awb_ref, gid_ref,
                 aw_ref, seg_ref):
    a, cc = _bn_coeffs(st_ref[...], g_ref[...], bb_ref[...])
    y = jnp.maximum(z_ref[...] * a + cc, 0.0)
    aw = jnp.sum(y * aww_ref[...], axis=1, keepdims=True) + awb_ref[...]
    aw_ref[...] = aw
    w = 1.0 / (1.0 + jnp.exp(-aw))
    hw = y * w
    oh = (gid_ref[...] == lax.broadcasted_iota(jnp.int32, (BLK, G), 1)
          ).astype(jnp.float32)

    @pl.when(pl.program_id(0) == 0)
    def _():
        seg_ref[...] = jnp.zeros((G, H), jnp.float32)

    seg_ref[...] += lax.dot_general(oh, hw, (((0,), (0,)), ((), ())),
                                    preferred_element_type=jnp.float32)


_kb_fin = pl.pallas_call(
    _kb_fin_body,
    grid=(NBLK,),
    in_specs=[
        pl.BlockSpec((BLK, H), lambda i: (i, 0)),
        pl.BlockSpec((2, H), lambda i: (0, 0)),
        pl.BlockSpec((1, H), lambda i: (0, 0)),
        pl.BlockSpec((1, H), lambda i: (0, 0)),
        pl.BlockSpec((1, H), lambda i: (0, 0)),
        pl.BlockSpec((1, 1), lambda i: (0, 0)),
        pl.BlockSpec((BLK, 1), lambda i: (i, 0)),
    ],
    out_specs=[
        pl.BlockSpec((BLK, 1), lambda i: (i, 0)),
        pl.BlockSpec((G, H), lambda i: (0, 0)),
    ],
    out_shape=[
        jax.ShapeDtypeStruct((N, 1), jnp.float32),
        jax.ShapeDtypeStruct((G, H), jnp.float32),
    ],
)


def _head_body(seg_ref, w1_ref, b1_ref, g1_ref, c1_ref, w2_ref, b2_ref,
               g2_ref, c2_ref, w3_ref, b3_ref, o_ref):
    x = jnp.dot(seg_ref[...], w1_ref[...],
                preferred_element_type=jnp.float32) + b1_ref[...]
    mu = jnp.mean(x, 0, keepdims=True)
    var = jnp.mean(x * x, 0, keepdims=True) - mu * mu
    x = jnp.maximum((x - mu) * lax.rsqrt(var + EPS) * g1_ref[...] + c1_ref[...],
                    0.0)
    x = jnp.dot(x, w2_ref[...], preferred_element_type=jnp.float32) + b2_ref[...]
    mu = jnp.mean(x, 0, keepdims=True)
    var = jnp.mean(x * x, 0, keepdims=True) - mu * mu
    x = jnp.maximum((x - mu) * lax.rsqrt(var + EPS) * g2_ref[...] + c2_ref[...],
                    0.0)
    x = jnp.dot(x, w3_ref[...], preferred_element_type=jnp.float32) + b3_ref[...]
    o_ref[...] = 1.0 / (1.0 + jnp.exp(-x))


_head = pl.pallas_call(
    _head_body,
    out_shape=jax.ShapeDtypeStruct((G, H), jnp.float32),
)


def kernel(feats, edge_index, node_graph_ids,
           gcn1_W, gcn1_b, gcn1_bn_g, gcn1_bn_b,
           gcn2_0_W, gcn2_0_b, gcn2_0_bn_g, gcn2_0_bn_b,
           gcn2_1_W, gcn2_1_b, gcn2_1_bn_g, gcn2_1_bn_b,
           aw_W, aw_b, fc1_W, fc1_b, bn1_g, bn1_b,
           lin0_W, lin0_b, bnl0_g, bnl0_b, fc2_W, fc2_b):
    src = edge_index[0]
    dst = edge_index[1]
    feats64 = jnp.pad(feats, ((0, 0), (0, INP - IN)))
    W1p = jnp.pad(gcn1_W, ((0, INP - IN), (0, 0)))
    gids2 = node_graph_ids.reshape(N, 1)

    degflat = _deg_kernel(src, dst)
    degr = degflat.reshape(2, 2, DEGP)
    po0 = degr[0, 0, :N].reshape(N, 1)
    pi0 = degr[0, 1, :N].reshape(N, 1)
    po1 = degr[1, 0, :N].reshape(N, 1)
    pi1 = degr[1, 1, :N].reshape(N, 1)

    t0a, t0b, t0c, t0d, ns, nd = _k0(feats64, po0, po1, pi0, pi1)

    u1 = _edge_scatter16(src, dst, t0a, t0b, t0c, t0d)
    z1, st1 = _ka64(*u1, nd, W1p, gcn1_b.reshape(1, H))
    t1 = _kb_mid(z1, st1, ns, gcn1_bn_g.reshape(1, H), gcn1_bn_b.reshape(1, H))

    u2 = _edge_scatter32(src, dst, *t1)
    z2, st2 = _ka128(*u2, nd, gcn2_0_W, gcn2_0_b.reshape(1, H))
    t2 = _kb_mid(z2, st2, ns, gcn2_0_bn_g.reshape(1, H),
                 gcn2_0_bn_b.reshape(1, H))

    u3 = _edge_scatter32(src, dst, *t2)
    z3, st3 = _ka128(*u3, nd, gcn2_1_W, gcn2_1_b.reshape(1, H))
    aw, seg = _kb_fin(z3, st3, gcn2_1_bn_g.reshape(1, H),
                      gcn2_1_bn_b.reshape(1, H), aw_W.reshape(1, H),
                      aw_b.reshape(1, 1), gids2)

    w3p = jnp.pad(fc2_W, ((0, 0), (0, H - OUT)))
    b3p = jnp.pad(fc2_b, ((0, H - OUT))).reshape(1, H)
    headp = _head(seg, fc1_W, fc1_b.reshape(1, 256), bn1_g.reshape(1, 256),
                  bn1_b.reshape(1, 256), lin0_W, lin0_b.reshape(1, H),
                  bnl0_g.reshape(1, H), bnl0_b.reshape(1, H), w3p, b3p)
    x = headp[:, :OUT]
    return (x, aw)
